# baseline (speedup 1.0000x reference)
import functools

import jax
import jax.numpy as jnp
from jax import lax
from jax.experimental import pallas as pl
from jax.experimental.pallas import tpu as pltpu
from jax.experimental.pallas import tpu_sc as plsc

N = 100000
NP = 100352
E = 6400000
NC = 2
NS = 16
NW = NC * NS
CHUNK = 544
NITER = 376
NIT2 = 748
EPW = CHUNK * NITER
E_PAD = EPW * NW
RPT = NP // NS
ZC = 448
RTC = NP // 32
F32 = jnp.float32


def _deg_body(dst_hbm, out_hbm, d0, d1, ones_v, zbuf_v, acc_sh, semi, sems):
    c = lax.axis_index("c")
    s = lax.axis_index("s")
    wid = s * NC + c
    zbase = s * RPT
    D = (d0, d1)

    def fill(i, carry):
        zbuf_v[pl.ds(i * 16, 16)] = jnp.zeros((16,), F32)
        return carry

    lax.fori_loop(0, ZC // 16, fill, 0)

    def fill1(i, carry):
        ones_v[pl.ds(i * 16, 16)] = jnp.ones((16,), F32)
        return carry

    lax.fori_loop(0, CHUNK // 16, fill1, 0)
    for j in range(RPT // ZC):
        pltpu.sync_copy(zbuf_v.at[pl.ds(0, ZC)],
                        acc_sh.at[pl.ds(zbase + j * ZC, ZC)])
    plsc.subcore_barrier()
    ebase = wid * EPW

    def window(w, b, first, prefetch):
        pltpu.make_async_copy(dst_hbm.at[pl.ds(ebase, CHUNK)], D[b], semi).wait()
        if not first:
            pltpu.make_async_copy(ones_v, acc_sh.at[D[0]], sems).wait()
        if prefetch:
            pltpu.async_copy(dst_hbm.at[pl.ds(ebase + (w + 1) * CHUNK, CHUNK)],
                             D[b ^ 1], semi)
        pltpu.async_copy(ones_v, acc_sh.at[D[b]], sems, add=True)

    pltpu.async_copy(dst_hbm.at[pl.ds(ebase, CHUNK)], D[0], semi)
    window(0, 0, True, True)

    def body(i2, carry):
        window(1 + 2 * i2, 1, False, True)
        window(2 + 2 * i2, 0, False, True)
        return carry

    lax.fori_loop(0, (NITER - 2) // 2, body, 0)
    window(NITER - 1, 1, False, False)
    pltpu.make_async_copy(ones_v, acc_sh.at[D[0]], sems).wait()
    plsc.subcore_barrier()
    for j in range(RPT // ZC):
        pltpu.sync_copy(acc_sh.at[pl.ds(zbase + j * ZC, ZC)],
                        zbuf_v.at[pl.ds(0, ZC)])
        pltpu.sync_copy(zbuf_v.at[pl.ds(0, ZC)],
                        out_hbm.at[pl.ds(c * NP + zbase + j * ZC, ZC)])


def _zero_acc(r0, acc_sh, zbase):
    def fill(i, carry):
        r0[i, :] = jnp.zeros((16,), F32)
        return carry

    lax.fori_loop(0, ZC, fill, 0)
    for j in range(RPT // ZC):
        pltpu.sync_copy(r0.at[pl.ds(0, ZC)],
                        acc_sh.at[pl.ds(zbase + j * ZC, ZC)])


def _copy_out(r0, acc_sh, out_hbm, zbase, c):
    for j in range(RPT // ZC):
        pltpu.sync_copy(acc_sh.at[pl.ds(zbase + j * ZC, ZC)],
                        r0.at[pl.ds(0, ZC)])
        pltpu.sync_copy(r0.at[pl.ds(0, ZC)],
                        out_hbm.at[pl.ds(c * NP + zbase + j * ZC, ZC)])


def _sweep(table_hbm, src_hbm, dst_hbm, S, D, R, acc_sh,
           semi, semg, sems, ebase, niter):
    def idx_fire(w, m4):
        off = ebase + w * CHUNK
        pltpu.async_copy(src_hbm.at[pl.ds(off, CHUNK)], S[m4], semi)
        pltpu.async_copy(dst_hbm.at[pl.ds(off, CHUNK)], D[m4], semi)

    def idx_wait(m4):
        pltpu.make_async_copy(src_hbm.at[pl.ds(ebase, CHUNK)], S[m4], semi).wait()
        pltpu.make_async_copy(dst_hbm.at[pl.ds(ebase, CHUNK)], D[m4], semi).wait()

    def s_wait():
        pltpu.make_async_copy(R[0], acc_sh.at[D[0]], sems).wait()

    def win(w, j, steady, pref, nxt):
        m3, m4 = j % 3, j % 4
        pltpu.make_async_copy(table_hbm.at[S[m4]], R[m3], semg).wait()
        pltpu.async_copy(R[m3], acc_sh.at[D[m4]], sems, add=True)
        if steady:
            s_wait()
        if pref:
            idx_fire(w + 2, (j + 2) % 4)
        if nxt:
            idx_wait((j + 1) % 4)
            pltpu.async_copy(table_hbm.at[S[(j + 1) % 4]], R[(j + 1) % 3],
                             semg)

    idx_fire(0, 0)
    idx_fire(1, 1)
    idx_wait(0)
    pltpu.async_copy(table_hbm.at[S[0]], R[0], semg)
    win(0, 0, False, True, True)
    win(1, 1, False, True, True)

    def body(i, carry):
        for j0 in range(12):
            win(2 + 12 * i + j0, 2 + j0, True, True, True)
        return carry

    lax.fori_loop(0, (niter - 4) // 12, body, 0)
    win(niter - 2, (niter - 2) % 12, True, False, True)
    win(niter - 1, (niter - 1) % 12, True, False, False)
    s_wait()
    s_wait()


def _prop_body(table_hbm, src_hbm, dst_hbm, out_hbm,
               s0, s1, s2, s3, d0, d1, d2, d3, r0, r1, r2,
               acc_sh, semi, semg, sems):
    c = lax.axis_index("c")
    s = lax.axis_index("s")
    zbase = s * RPT
    _zero_acc(r0, acc_sh, zbase)
    plsc.subcore_barrier()
    _sweep(table_hbm, src_hbm, dst_hbm, (s0, s1, s2, s3), (d0, d1, d2, d3),
           (r0, r1, r2), acc_sh, semi, semg, sems, (s * NC + c) * EPW, NITER)
    plsc.subcore_barrier()
    _copy_out(r0, acc_sh, out_hbm, zbase, c)


def _prop2_body(lo_hbm, hi_hbm, src_hbm, dst_hbm, out_hbm,
                s0, s1, s2, s3, d0, d1, d2, d3, r0, r1, r2,
                acc_sh, semi, semg, sems):
    c = lax.axis_index("c")
    s = lax.axis_index("s")
    zbase = s * RPT
    _zero_acc(r0, acc_sh, zbase)
    plsc.subcore_barrier()
    ebase = s * (CHUNK * NIT2)

    @pl.when(c == 0)
    def _():
        _sweep(lo_hbm, src_hbm, dst_hbm, (s0, s1, s2, s3), (d0, d1, d2, d3),
               (r0, r1, r2), acc_sh, semi, semg, sems, ebase, NIT2)

    @pl.when(c == 1)
    def _():
        _sweep(hi_hbm, src_hbm, dst_hbm, (s0, s1, s2, s3), (d0, d1, d2, d3),
               (r0, r1, r2), acc_sh, semi, semg, sems, ebase, NIT2)

    plsc.subcore_barrier()
    _copy_out(r0, acc_sh, out_hbm, zbase, c)


@functools.cache
def _sc_kernels():
    mesh = plsc.VectorSubcoreMesh(core_axis_name="c", subcore_axis_name="s",
                                  num_cores=NC, num_subcores=NS)
    params = pltpu.CompilerParams(use_tc_tiling_on_sc=False)
    deg = pl.kernel(
        _deg_body,
        out_type=jax.ShapeDtypeStruct((NC * NP,), F32),
        mesh=mesh,
        compiler_params=params,
        scratch_types=[
            pltpu.VMEM((CHUNK,), jnp.int32),
            pltpu.VMEM((CHUNK,), jnp.int32),
            pltpu.VMEM((CHUNK,), F32),
            pltpu.VMEM((ZC,), F32),
            pltpu.VMEM_SHARED((NP,), F32),
            pltpu.SemaphoreType.DMA,
            pltpu.SemaphoreType.DMA,
        ],
    )
    prop_scratch = (
        [pltpu.VMEM((CHUNK,), jnp.int32)] * 8
        + [pltpu.VMEM((CHUNK, 16), F32)] * 3
        + [
            pltpu.VMEM_SHARED((NP, 16), F32),
            pltpu.SemaphoreType.DMA,
            pltpu.SemaphoreType.DMA,
            pltpu.SemaphoreType.DMA,
        ]
    )
    prop = pl.kernel(
        _prop_body,
        out_type=jax.ShapeDtypeStruct((NC * NP, 16), F32),
        mesh=mesh,
        compiler_params=params,
        scratch_types=prop_scratch,
    )
    prop2 = pl.kernel(
        _prop2_body,
        out_type=jax.ShapeDtypeStruct((NC * NP, 16), F32),
        mesh=mesh,
        compiler_params=params,
        scratch_types=prop_scratch,
    )
    return deg, prop, prop2


def _tc1_body(degp_ref, x_ref, dinv_ref, xs_ref):
    deg = degp_ref[0] + degp_ref[1] + 1.0
    dinv = lax.rsqrt(deg)
    dinv_ref[...] = dinv
    xs_ref[...] = x_ref[...] * dinv


def _tc1(deg_part, x_pad):
    return pl.pallas_call(
        _tc1_body,
        grid=(32,),
        in_specs=[
            pl.BlockSpec((2, RTC, 1), lambda i: (0, i, 0)),
            pl.BlockSpec((RTC, 16), lambda i: (i, 0)),
        ],
        out_specs=[
            pl.BlockSpec((RTC, 1), lambda i: (i, 0)),
            pl.BlockSpec((RTC, 16), lambda i: (i, 0)),
        ],
        out_shape=[
            jax.ShapeDtypeStruct((NP, 1), F32),
            jax.ShapeDtypeStruct((NP, 16), F32),
        ],
    )(deg_part, x_pad)


def _tc2_body(s1_ref, xs_ref, dinv_ref, w1_ref, b1_ref, w2_ref, lo_ref, hi_ref):
    dinv = dinv_ref[...]
    m1 = dinv * (s1_ref[0] + s1_ref[1] + xs_ref[...])
    h1 = jnp.maximum(
        jnp.dot(m1, w1_ref[...], preferred_element_type=F32) + b1_ref[...], 0.0)
    ps = dinv * jnp.dot(h1, w2_ref[...], preferred_element_type=F32)
    lo_ref[...] = ps[:, :16]
    hi_ref[...] = ps[:, 16:]


def _tc2(s1, xs, dinv, w1p, b1r, w2):
    return pl.pallas_call(
        _tc2_body,
        grid=(32,),
        in_specs=[
            pl.BlockSpec((2, RTC, 16), lambda i: (0, i, 0)),
            pl.BlockSpec((RTC, 16), lambda i: (i, 0)),
            pl.BlockSpec((RTC, 1), lambda i: (i, 0)),
            pl.BlockSpec((16, 64), lambda i: (0, 0)),
            pl.BlockSpec((1, 64), lambda i: (0, 0)),
            pl.BlockSpec((64, 32), lambda i: (0, 0)),
        ],
        out_specs=[
            pl.BlockSpec((RTC, 16), lambda i: (i, 0)),
            pl.BlockSpec((RTC, 16), lambda i: (i, 0)),
        ],
        out_shape=[
            jax.ShapeDtypeStruct((NP, 16), F32),
            jax.ShapeDtypeStruct((NP, 16), F32),
        ],
    )(s1, xs, dinv, w1p, b1r, w2)


def _tc3_body(s2_ref, pslo_ref, pshi_ref, dinv_ref,
              b2_ref, wout_ref, bout_ref, out_ref):
    i = pl.program_id(0)
    dinv = dinv_ref[...]
    m2lo = dinv * (s2_ref[0] + pslo_ref[...])
    m2hi = dinv * (s2_ref[1] + pshi_ref[...])
    m2 = jnp.concatenate([m2lo, m2hi], axis=1)
    h2 = jnp.maximum(m2 + b2_ref[...], 0.0)
    rows = i * RTC + lax.broadcasted_iota(jnp.int32, (RTC, 1), 0)
    h2 = jnp.where(rows < N, h2, 0.0)
    part = jnp.sum(jnp.dot(h2, wout_ref[...], preferred_element_type=F32))

    @pl.when(i == 0)
    def _():
        out_ref[...] = jnp.zeros((1, 1), F32)

    out_ref[...] = out_ref[...] + part

    @pl.when(i == 31)
    def _():
        out_ref[...] = out_ref[...] / jnp.float32(N) + bout_ref[...]


def _tc3(s2, ps_lo, ps_hi, dinv, b2r, wout, boutr):
    return pl.pallas_call(
        _tc3_body,
        grid=(32,),
        in_specs=[
            pl.BlockSpec((2, RTC, 16), lambda i: (0, i, 0)),
            pl.BlockSpec((RTC, 16), lambda i: (i, 0)),
            pl.BlockSpec((RTC, 16), lambda i: (i, 0)),
            pl.BlockSpec((RTC, 1), lambda i: (i, 0)),
            pl.BlockSpec((1, 32), lambda i: (0, 0)),
            pl.BlockSpec((32, 1), lambda i: (0, 0)),
            pl.BlockSpec((1, 1), lambda i: (0, 0)),
        ],
        out_specs=pl.BlockSpec((1, 1), lambda i: (0, 0)),
        out_shape=jax.ShapeDtypeStruct((1, 1), F32),
    )(s2, ps_lo, ps_hi, dinv, b2r, wout, boutr)


def kernel(x, edge_index, W1, b1, W2, b2, Wout, bout):
    pad = N + (jnp.arange(E_PAD - E, dtype=jnp.int32) % (NP - N))
    src = jnp.concatenate([edge_index[0].astype(jnp.int32), pad])
    dst = jnp.concatenate([edge_index[1].astype(jnp.int32), pad])
    x_pad = jnp.zeros((NP, 16), F32).at[:N, :9].set(x)
    w1p = jnp.zeros((16, 64), F32).at[:9].set(W1)

    _deg_sc, _prop_sc, _prop2_sc = _sc_kernels()
    deg_part = _deg_sc(dst)
    dinv, xs = _tc1(deg_part.reshape(NC, NP, 1), x_pad)
    s1 = _prop_sc(xs, src, dst).reshape(NC, NP, 16)
    ps_lo, ps_hi = _tc2(s1, xs, dinv, w1p, b1.reshape(1, 64), W2)
    s2 = _prop2_sc(ps_lo, ps_hi, src, dst).reshape(NC, NP, 16)
    out = _tc3(s2, ps_lo, ps_hi, dinv,
               b2.reshape(1, 32), Wout, bout.reshape(1, 1))
    return out[:, 0]


# speedup vs baseline: 1.1662x; 1.1662x over previous
import functools

import jax
import jax.numpy as jnp
from jax import lax
from jax.experimental import pallas as pl
from jax.experimental.pallas import tpu as pltpu
from jax.experimental.pallas import tpu_sc as plsc

N = 100000
NP = 100352
E = 6400000
NC = 2
NS = 16
NW = NC * NS
CHUNK = 848
NITER = 236
NIT2 = NITER * 2
EPW = CHUNK * NITER
E_PAD = EPW * NW
RPT = NP // NS
ZC = 784
RTC = NP // 32
F32 = jnp.float32


def _deg_body(dst_hbm, out_hbm, d0, d1, ones_v, zbuf_v, acc_sh, semi, sems):
    c = lax.axis_index("c")
    s = lax.axis_index("s")
    wid = s * NC + c
    zbase = s * RPT
    D = (d0, d1)

    def fill(i, carry):
        zbuf_v[pl.ds(i * 16, 16)] = jnp.zeros((16,), F32)
        return carry

    lax.fori_loop(0, ZC // 16, fill, 0)

    def fill1(i, carry):
        ones_v[pl.ds(i * 16, 16)] = jnp.ones((16,), F32)
        return carry

    lax.fori_loop(0, CHUNK // 16, fill1, 0)
    for j in range(RPT // ZC):
        pltpu.sync_copy(zbuf_v.at[pl.ds(0, ZC)],
                        acc_sh.at[pl.ds(zbase + j * ZC, ZC)])
    plsc.subcore_barrier()
    ebase = wid * EPW

    def window(w, b, first, prefetch):
        pltpu.make_async_copy(dst_hbm.at[pl.ds(ebase, CHUNK)], D[b], semi).wait()
        if not first:
            pltpu.make_async_copy(ones_v, acc_sh.at[D[0]], sems).wait()
        if prefetch:
            pltpu.async_copy(dst_hbm.at[pl.ds(ebase + (w + 1) * CHUNK, CHUNK)],
                             D[b ^ 1], semi)
        pltpu.async_copy(ones_v, acc_sh.at[D[b]], sems, add=True)

    pltpu.async_copy(dst_hbm.at[pl.ds(ebase, CHUNK)], D[0], semi)
    window(0, 0, True, True)

    def body(i2, carry):
        window(1 + 2 * i2, 1, False, True)
        window(2 + 2 * i2, 0, False, True)
        return carry

    lax.fori_loop(0, (NITER - 2) // 2, body, 0)
    window(NITER - 1, 1, False, False)
    pltpu.make_async_copy(ones_v, acc_sh.at[D[0]], sems).wait()
    plsc.subcore_barrier()
    for j in range(RPT // ZC):
        pltpu.sync_copy(acc_sh.at[pl.ds(zbase + j * ZC, ZC)],
                        zbuf_v.at[pl.ds(0, ZC)])
        pltpu.sync_copy(zbuf_v.at[pl.ds(0, ZC)],
                        out_hbm.at[pl.ds(c * NP + zbase + j * ZC, ZC)])


def _zero_acc(r0, acc_sh, zbase):
    def fill(i, carry):
        r0[i, :] = jnp.zeros((16,), F32)
        return carry

    lax.fori_loop(0, ZC, fill, 0)
    for j in range(RPT // ZC):
        pltpu.sync_copy(r0.at[pl.ds(0, ZC)],
                        acc_sh.at[pl.ds(zbase + j * ZC, ZC)])


def _copy_out(r0, acc_sh, out_hbm, zbase, c):
    for j in range(RPT // ZC):
        pltpu.sync_copy(acc_sh.at[pl.ds(zbase + j * ZC, ZC)],
                        r0.at[pl.ds(0, ZC)])
        pltpu.sync_copy(r0.at[pl.ds(0, ZC)],
                        out_hbm.at[pl.ds(c * NP + zbase + j * ZC, ZC)])


def _sweep(table_hbm, src_hbm, dst_hbm, S, D, R, acc_sh,
           semi, semg, sems, ebase, niter):
    def window(w, b, first, prefetch):
        pltpu.make_async_copy(src_hbm.at[pl.ds(ebase, CHUNK)], S[b], semi).wait()
        pltpu.make_async_copy(dst_hbm.at[pl.ds(ebase, CHUNK)], D[b], semi).wait()
        g = pltpu.async_copy(table_hbm.at[S[b]], R[b], semg)
        if not first:
            pltpu.make_async_copy(R[0], acc_sh.at[D[0]], sems).wait()
        if prefetch:
            off = ebase + (w + 1) * CHUNK
            pltpu.async_copy(src_hbm.at[pl.ds(off, CHUNK)], S[b ^ 1], semi)
            pltpu.async_copy(dst_hbm.at[pl.ds(off, CHUNK)], D[b ^ 1], semi)
        g.wait()
        pltpu.async_copy(R[b], acc_sh.at[D[b]], sems, add=True)

    pltpu.async_copy(src_hbm.at[pl.ds(ebase, CHUNK)], S[0], semi)
    pltpu.async_copy(dst_hbm.at[pl.ds(ebase, CHUNK)], D[0], semi)
    window(0, 0, True, True)

    def body(i2, carry):
        window(1 + 2 * i2, 1, False, True)
        window(2 + 2 * i2, 0, False, True)
        return carry

    lax.fori_loop(0, (niter - 2) // 2, body, 0)
    window(niter - 1, 1, False, False)
    pltpu.make_async_copy(R[0], acc_sh.at[D[0]], sems).wait()


def _prop_body(table_hbm, src_hbm, dst_hbm, out_hbm,
               s0, s1, d0, d1, r0, r1, acc_sh, semi, semg, sems):
    c = lax.axis_index("c")
    s = lax.axis_index("s")
    zbase = s * RPT
    _zero_acc(r0, acc_sh, zbase)
    plsc.subcore_barrier()
    _sweep(table_hbm, src_hbm, dst_hbm, (s0, s1), (d0, d1), (r0, r1),
           acc_sh, semi, semg, sems, (s * NC + c) * EPW, NITER)
    plsc.subcore_barrier()
    _copy_out(r0, acc_sh, out_hbm, zbase, c)


def _prop2_body(lo_hbm, hi_hbm, src_hbm, dst_hbm, out_hbm,
                s0, s1, d0, d1, r0, r1, acc_sh, semi, semg, sems):
    c = lax.axis_index("c")
    s = lax.axis_index("s")
    zbase = s * RPT
    _zero_acc(r0, acc_sh, zbase)
    plsc.subcore_barrier()
    ebase = s * (CHUNK * NIT2)

    @pl.when(c == 0)
    def _():
        _sweep(lo_hbm, src_hbm, dst_hbm, (s0, s1), (d0, d1), (r0, r1),
               acc_sh, semi, semg, sems, ebase, NIT2)

    @pl.when(c == 1)
    def _():
        _sweep(hi_hbm, src_hbm, dst_hbm, (s0, s1), (d0, d1), (r0, r1),
               acc_sh, semi, semg, sems, ebase, NIT2)

    plsc.subcore_barrier()
    _copy_out(r0, acc_sh, out_hbm, zbase, c)


@functools.cache
def _sc_kernels():
    mesh = plsc.VectorSubcoreMesh(core_axis_name="c", subcore_axis_name="s",
                                  num_cores=NC, num_subcores=NS)
    params = pltpu.CompilerParams(use_tc_tiling_on_sc=False)
    deg = pl.kernel(
        _deg_body,
        out_type=jax.ShapeDtypeStruct((NC * NP,), F32),
        mesh=mesh,
        compiler_params=params,
        scratch_types=[
            pltpu.VMEM((CHUNK,), jnp.int32),
            pltpu.VMEM((CHUNK,), jnp.int32),
            pltpu.VMEM((CHUNK,), F32),
            pltpu.VMEM((ZC,), F32),
            pltpu.VMEM_SHARED((NP,), F32),
            pltpu.SemaphoreType.DMA,
            pltpu.SemaphoreType.DMA,
        ],
    )
    prop_scratch = [
        pltpu.VMEM((CHUNK,), jnp.int32),
        pltpu.VMEM((CHUNK,), jnp.int32),
        pltpu.VMEM((CHUNK,), jnp.int32),
        pltpu.VMEM((CHUNK,), jnp.int32),
        pltpu.VMEM((CHUNK, 16), F32),
        pltpu.VMEM((CHUNK, 16), F32),
        pltpu.VMEM_SHARED((NP, 16), F32),
        pltpu.SemaphoreType.DMA,
        pltpu.SemaphoreType.DMA,
        pltpu.SemaphoreType.DMA,
    ]
    prop = pl.kernel(
        _prop_body,
        out_type=jax.ShapeDtypeStruct((NC * NP, 16), F32),
        mesh=mesh,
        compiler_params=params,
        scratch_types=prop_scratch,
    )
    prop2 = pl.kernel(
        _prop2_body,
        out_type=jax.ShapeDtypeStruct((NC * NP, 16), F32),
        mesh=mesh,
        compiler_params=params,
        scratch_types=prop_scratch,
    )
    return deg, prop, prop2


def _tc1_body(degp_ref, x_ref, dinv_ref, xs_ref):
    deg = degp_ref[0] + degp_ref[1] + 1.0
    dinv = lax.rsqrt(deg)
    dinv_ref[...] = dinv
    xs_ref[...] = x_ref[...] * dinv


def _tc1(deg_part, x_pad):
    return pl.pallas_call(
        _tc1_body,
        grid=(32,),
        in_specs=[
            pl.BlockSpec((2, RTC, 1), lambda i: (0, i, 0)),
            pl.BlockSpec((RTC, 16), lambda i: (i, 0)),
        ],
        out_specs=[
            pl.BlockSpec((RTC, 1), lambda i: (i, 0)),
            pl.BlockSpec((RTC, 16), lambda i: (i, 0)),
        ],
        out_shape=[
            jax.ShapeDtypeStruct((NP, 1), F32),
            jax.ShapeDtypeStruct((NP, 16), F32),
        ],
    )(deg_part, x_pad)


def _tc2_body(s1_ref, xs_ref, dinv_ref, w1_ref, b1_ref, w2_ref, lo_ref, hi_ref):
    dinv = dinv_ref[...]
    m1 = dinv * (s1_ref[0] + s1_ref[1] + xs_ref[...])
    h1 = jnp.maximum(
        jnp.dot(m1, w1_ref[...], preferred_element_type=F32) + b1_ref[...], 0.0)
    ps = dinv * jnp.dot(h1, w2_ref[...], preferred_element_type=F32)
    lo_ref[...] = ps[:, :16]
    hi_ref[...] = ps[:, 16:]


def _tc2(s1, xs, dinv, w1p, b1r, w2):
    return pl.pallas_call(
        _tc2_body,
        grid=(32,),
        in_specs=[
            pl.BlockSpec((2, RTC, 16), lambda i: (0, i, 0)),
            pl.BlockSpec((RTC, 16), lambda i: (i, 0)),
            pl.BlockSpec((RTC, 1), lambda i: (i, 0)),
            pl.BlockSpec((16, 64), lambda i: (0, 0)),
            pl.BlockSpec((1, 64), lambda i: (0, 0)),
            pl.BlockSpec((64, 32), lambda i: (0, 0)),
        ],
        out_specs=[
            pl.BlockSpec((RTC, 16), lambda i: (i, 0)),
            pl.BlockSpec((RTC, 16), lambda i: (i, 0)),
        ],
        out_shape=[
            jax.ShapeDtypeStruct((NP, 16), F32),
            jax.ShapeDtypeStruct((NP, 16), F32),
        ],
    )(s1, xs, dinv, w1p, b1r, w2)


def _tc3_body(s2_ref, pslo_ref, pshi_ref, dinv_ref,
              b2_ref, wout_ref, bout_ref, out_ref):
    i = pl.program_id(0)
    dinv = dinv_ref[...]
    m2lo = dinv * (s2_ref[0] + pslo_ref[...])
    m2hi = dinv * (s2_ref[1] + pshi_ref[...])
    m2 = jnp.concatenate([m2lo, m2hi], axis=1)
    h2 = jnp.maximum(m2 + b2_ref[...], 0.0)
    rows = i * RTC + lax.broadcasted_iota(jnp.int32, (RTC, 1), 0)
    h2 = jnp.where(rows < N, h2, 0.0)
    part = jnp.sum(jnp.dot(h2, wout_ref[...], preferred_element_type=F32))

    @pl.when(i == 0)
    def _():
        out_ref[...] = jnp.zeros((1, 1), F32)

    out_ref[...] = out_ref[...] + part

    @pl.when(i == 31)
    def _():
        out_ref[...] = out_ref[...] / jnp.float32(N) + bout_ref[...]


def _tc3(s2, ps_lo, ps_hi, dinv, b2r, wout, boutr):
    return pl.pallas_call(
        _tc3_body,
        grid=(32,),
        in_specs=[
            pl.BlockSpec((2, RTC, 16), lambda i: (0, i, 0)),
            pl.BlockSpec((RTC, 16), lambda i: (i, 0)),
            pl.BlockSpec((RTC, 16), lambda i: (i, 0)),
            pl.BlockSpec((RTC, 1), lambda i: (i, 0)),
            pl.BlockSpec((1, 32), lambda i: (0, 0)),
            pl.BlockSpec((32, 1), lambda i: (0, 0)),
            pl.BlockSpec((1, 1), lambda i: (0, 0)),
        ],
        out_specs=pl.BlockSpec((1, 1), lambda i: (0, 0)),
        out_shape=jax.ShapeDtypeStruct((1, 1), F32),
    )(s2, ps_lo, ps_hi, dinv, b2r, wout, boutr)


def kernel(x, edge_index, W1, b1, W2, b2, Wout, bout):
    pad = N + (jnp.arange(E_PAD - E, dtype=jnp.int32) % (NP - N))
    src = jnp.concatenate([edge_index[0].astype(jnp.int32), pad])
    dst = jnp.concatenate([edge_index[1].astype(jnp.int32), pad])
    x_pad = jnp.zeros((NP, 16), F32).at[:N, :9].set(x)
    w1p = jnp.zeros((16, 64), F32).at[:9].set(W1)

    _deg_sc, _prop_sc, _prop2_sc = _sc_kernels()
    deg_part = _deg_sc(dst)
    dinv, xs = _tc1(deg_part.reshape(NC, NP, 1), x_pad)
    s1 = _prop_sc(xs, src, dst).reshape(NC, NP, 16)
    ps_lo, ps_hi = _tc2(s1, xs, dinv, w1p, b1.reshape(1, 64), W2)
    s2 = _prop2_sc(ps_lo, ps_hi, src, dst).reshape(NC, NP, 16)
    out = _tc3(s2, ps_lo, ps_hi, dinv,
               b2.reshape(1, 32), Wout, bout.reshape(1, 1))
    return out[:, 0]
